# baseline (speedup 1.0000x reference)
import functools

import jax
import jax.numpy as jnp
from jax import lax
from jax.experimental import pallas as pl
from jax.experimental.pallas import tpu as pltpu
from jax.experimental.pallas import tpu_sc as plsc

H = 1_000_000
ALPHA = 0.01
L = 16
IDX_CHUNK = 512


def _make_sc_kernel(batch, num_workers):
    n_per_w = batch // num_workers
    n_chunks = n_per_w // IDX_CHUNK
    mesh = plsc.VectorSubcoreMesh(core_axis_name="c", subcore_axis_name="s")

    @functools.partial(
        pl.kernel,
        mesh=mesh,
        out_type=jax.ShapeDtypeStruct((batch,), jnp.float32),
        scratch_types=[
            pltpu.VMEM((n_per_w,), jnp.int32),
            pltpu.VMEM((n_chunks, IDX_CHUNK), jnp.int32),
            pltpu.VMEM((n_per_w,), jnp.float32),
            pltpu.VMEM((n_per_w,), jnp.float32),
            pltpu.VMEM((L,), jnp.float32),
            pltpu.VMEM((n_per_w,), jnp.float32),
            pltpu.SemaphoreType.DMA,
            pltpu.SemaphoreType.DMA,
            pltpu.SemaphoreType.DMA,
        ],
    )
    def sc_kernel(ids_hbm, a_hbm, b_hbm, step_hbm, out_hbm,
                  ids_v, q_v, a_v, b_v, step_v, o_v, sem_a, sem_b, sem_in):
        nc = plsc.get_sparse_core_info().num_cores
        wid = lax.axis_index("s") * nc + lax.axis_index("c")
        base = wid * n_per_w

        cp_ids = pltpu.async_copy(ids_hbm.at[pl.ds(base, n_per_w)], ids_v,
                                  sem_in)
        cp_step = pltpu.async_copy(step_hbm, step_v, sem_in)
        cp_ids.wait()

        copies = []
        for j in range(n_chunks):
            for i in range(IDX_CHUNK // L):
                ids16 = ids_v[pl.ds(j * IDX_CHUNK + i * L, L)]
                q_v[j, pl.ds(i * L, L)] = lax.rem(ids16, jnp.int32(H))
            copies.append(pltpu.async_copy(
                a_hbm.at[q_v.at[j]], a_v.at[pl.ds(j * IDX_CHUNK, IDX_CHUNK)],
                sem_a))
            copies.append(pltpu.async_copy(
                b_hbm.at[q_v.at[j]], b_v.at[pl.ds(j * IDX_CHUNK, IDX_CHUNK)],
                sem_b))
        for cp in copies:
            cp.wait()
        cp_step.wait()

        step16 = step_v[...]
        for i in range(n_per_w // L):
            a16 = a_v[pl.ds(i * L, L)]
            b16 = b_v[pl.ds(i * L, L)]
            denom = (1.0 - ALPHA) * b16 + ALPHA * (step16 - a16)
            o_v[pl.ds(i * L, L)] = 1.0 / (denom + 1e-8)

        pltpu.sync_copy(o_v, out_hbm.at[pl.ds(base, n_per_w)])

    return sc_kernel


def kernel(batch_item_ids, A, B, step):
    batch = batch_item_ids.shape[0]
    info = plsc.get_sparse_core_info()
    num_workers = info.num_cores * info.num_subcores
    step_vec = jnp.full((L,), step, dtype=jnp.float32)
    sc = _make_sc_kernel(batch, num_workers)
    return sc(batch_item_ids, A, B, step_vec)


# speedup vs baseline: 1.3288x; 1.3288x over previous
import functools

import jax
import jax.numpy as jnp
from jax import lax
from jax.experimental import pallas as pl
from jax.experimental.pallas import tpu as pltpu
from jax.experimental.pallas import tpu_sc as plsc

H = 1_000_000
ALPHA = 0.01
L = 16
IDX_CHUNK = 128


def _make_sc_kernel(batch, num_workers):
    n_per_w = batch // num_workers
    n_chunks = n_per_w // IDX_CHUNK
    mesh = plsc.VectorSubcoreMesh(core_axis_name="c", subcore_axis_name="s")

    @functools.partial(
        pl.kernel,
        mesh=mesh,
        out_type=jax.ShapeDtypeStruct((batch,), jnp.float32),
        scratch_types=[
            pltpu.VMEM((n_per_w,), jnp.int32),
            pltpu.VMEM((n_chunks, IDX_CHUNK), jnp.int32),
            pltpu.VMEM((n_per_w,), jnp.float32),
            pltpu.VMEM((n_per_w,), jnp.float32),
            pltpu.VMEM((L,), jnp.float32),
            pltpu.VMEM((n_per_w,), jnp.float32),
            pltpu.SemaphoreType.DMA,
            pltpu.SemaphoreType.DMA,
            pltpu.SemaphoreType.DMA,
        ],
    )
    def sc_kernel(ids_hbm, a_hbm, b_hbm, step_hbm, out_hbm,
                  ids_v, q_v, a_v, b_v, step_v, o_v, sem_a, sem_b, sem_in):
        nc = plsc.get_sparse_core_info().num_cores
        wid = lax.axis_index("s") * nc + lax.axis_index("c")
        base = wid * n_per_w

        cp_ids = pltpu.async_copy(ids_hbm.at[pl.ds(base, n_per_w)], ids_v,
                                  sem_in)
        cp_step = pltpu.async_copy(step_hbm, step_v, sem_in)
        cp_ids.wait()

        copies = []
        for j in range(n_chunks):
            for i in range(IDX_CHUNK // L):
                ids16 = ids_v[pl.ds(j * IDX_CHUNK + i * L, L)]
                lo = jnp.bitwise_and(ids16, jnp.int32(0xFFFF))
                pf = (ids16 - lo).astype(jnp.float32)
                k = (pf * jnp.float32(1e-6)).astype(jnp.int32)
                t = pf - k.astype(jnp.float32) * jnp.float32(H)
                r0 = t + lo.astype(jnp.float32)
                r = jnp.where(r0 >= jnp.float32(H), r0 - jnp.float32(H), r0)
                q_v[j, pl.ds(i * L, L)] = r.astype(jnp.int32)
            copies.append(pltpu.async_copy(
                a_hbm.at[q_v.at[j]], a_v.at[pl.ds(j * IDX_CHUNK, IDX_CHUNK)],
                sem_a))
            copies.append(pltpu.async_copy(
                b_hbm.at[q_v.at[j]], b_v.at[pl.ds(j * IDX_CHUNK, IDX_CHUNK)],
                sem_b))
        for cp in copies:
            cp.wait()
        cp_step.wait()

        step16 = step_v[...]
        for i in range(n_per_w // L):
            a16 = a_v[pl.ds(i * L, L)]
            b16 = b_v[pl.ds(i * L, L)]
            denom = (1.0 - ALPHA) * b16 + ALPHA * (step16 - a16)
            o_v[pl.ds(i * L, L)] = 1.0 / (denom + 1e-8)

        pltpu.sync_copy(o_v, out_hbm.at[pl.ds(base, n_per_w)])

    return sc_kernel


def kernel(batch_item_ids, A, B, step):
    batch = batch_item_ids.shape[0]
    info = plsc.get_sparse_core_info()
    num_workers = info.num_cores * info.num_subcores
    step_vec = jnp.full((L,), step, dtype=jnp.float32)
    sc = _make_sc_kernel(batch, num_workers)
    return sc(batch_item_ids, A, B, step_vec)
